# initial kernel scaffold (unmeasured)
import jax
import jax.numpy as jnp
from jax import lax
from jax.experimental import pallas as pl
from jax.experimental.pallas import tpu as pltpu

N_DEV = 8
B = 2
SQ = 512
SKV = 512
H_LOC = 8
DH = 64
D_MODEL = 768
HD_LOC = H_LOC * DH


def kernel(x, Wq, K_ext, V_ext, Wo):
    my = lax.axis_index("i")

    Wq_loc = lax.dynamic_slice_in_dim(Wq, my * HD_LOC, HD_LOC, axis=1)
    Wo_loc = lax.dynamic_slice_in_dim(Wo, my * HD_LOC, HD_LOC, axis=0)
    Kt = jnp.transpose(K_ext, (0, 2, 1, 3))
    Vt = jnp.transpose(V_ext, (0, 2, 1, 3))

    def body(x_ref, wq_ref, k_ref, v_ref, wo_ref, out_ref,
             q_ref, ctx_ref, comm_ref, send_sems, recv_sems):
        left = (my - 1) % N_DEV
        right = (my + 1) % N_DEV

        barrier_sem = pltpu.get_barrier_semaphore()
        for nbr in [left, right]:
            pl.semaphore_signal(
                barrier_sem, inc=1,
                device_id=(nbr,), device_id_type=pl.DeviceIdType.MESH,
            )
        pl.semaphore_wait(barrier_sem, 2)

        xv = x_ref[:].reshape(B * SQ, D_MODEL)
        q_ref[:] = jnp.dot(xv, wq_ref[:], preferred_element_type=jnp.float32)

        qb = lax.broadcasted_iota(jnp.int32, (SQ, SKV), 0) // 64
        kb = lax.broadcasted_iota(jnp.int32, (SQ, SKV), 1) // 64
        mask = (qb == kb) | (kb == 0) | ((qb + kb) % 3 == 0)

        for b in range(B):
            for h in range(H_LOC):
                qh = q_ref[b * SQ:(b + 1) * SQ, h * DH:(h + 1) * DH]
                kh = k_ref[b, h]
                scores = lax.dot_general(
                    qh, kh, (((1,), (1,)), ((), ())),
                    preferred_element_type=jnp.float32,
                ) * 0.125
                scores = jnp.where(mask, scores, -1e9)
                m = jnp.max(scores, axis=1, keepdims=True)
                e = jnp.exp(scores - m)
                w = e / jnp.sum(e, axis=1, keepdims=True)
                ctx_ref[:, h * DH:(h + 1) * DH] = jnp.dot(
                    w, v_ref[b, h], preferred_element_type=jnp.float32
                )
            partial_b = jnp.dot(
                ctx_ref[:], wo_ref[:], preferred_element_type=jnp.float32
            )
            out_ref[b] = partial_b
            comm_ref[0, b] = partial_b

        for h in range(N_DEV - 1):
            send_slot = h % 2
            recv_slot = (h + 1) % 2
            rdma = pltpu.make_async_remote_copy(
                src_ref=comm_ref.at[send_slot],
                dst_ref=comm_ref.at[recv_slot],
                send_sem=send_sems.at[send_slot],
                recv_sem=recv_sems.at[recv_slot],
                device_id=(right,),
                device_id_type=pl.DeviceIdType.MESH,
            )
            rdma.start()
            rdma.wait()
            out_ref[:] = out_ref[:] + comm_ref[recv_slot]

    return pl.pallas_call(
        body,
        out_shape=jax.ShapeDtypeStruct((B, SQ, D_MODEL), jnp.float32),
        in_specs=[pl.BlockSpec(memory_space=pltpu.VMEM)] * 5,
        out_specs=pl.BlockSpec(memory_space=pltpu.VMEM),
        scratch_shapes=[
            pltpu.VMEM((B * SQ, HD_LOC), jnp.float32),
            pltpu.VMEM((SQ, HD_LOC), jnp.float32),
            pltpu.VMEM((2, B, SQ, D_MODEL), jnp.float32),
            pltpu.SemaphoreType.DMA((2,)),
            pltpu.SemaphoreType.DMA((2,)),
        ],
        compiler_params=pltpu.CompilerParams(collective_id=0),
    )(x, Wq_loc, Kt, Vt, Wo_loc)


# baseline (device time: 273492 ns/iter reference)
import jax
import jax.numpy as jnp
from jax import lax
from jax.experimental import pallas as pl
from jax.experimental.pallas import tpu as pltpu

N_DEV = 8
B = 2
SQ = 512
SKV = 512
H_LOC = 8
DH = 64
D_MODEL = 768
HD_LOC = H_LOC * DH


def kernel(x, Wq, K_ext, V_ext, Wo):
    my = lax.axis_index("i")

    Wq_loc = lax.dynamic_slice_in_dim(Wq, my * HD_LOC, HD_LOC, axis=1)
    Wo_loc = lax.dynamic_slice_in_dim(Wo, my * HD_LOC, HD_LOC, axis=0)
    Kt = jnp.transpose(K_ext, (0, 2, 1, 3))
    Vt = jnp.transpose(V_ext, (0, 2, 1, 3))

    def body(x_ref, wq_ref, k_ref, v_ref, wo_ref, out_ref,
             q_ref, ctx_ref, comm_ref, send_sems, recv_sems):
        my_pos = lax.axis_index("i")
        left = (my_pos - 1) % N_DEV
        right = (my_pos + 1) % N_DEV

        barrier_sem = pltpu.get_barrier_semaphore()
        for nbr in [left, right]:
            pl.semaphore_signal(
                barrier_sem, inc=1,
                device_id=(nbr,), device_id_type=pl.DeviceIdType.MESH,
            )
        pl.semaphore_wait(barrier_sem, 2)

        xv = x_ref[:].reshape(B * SQ, D_MODEL)
        q_ref[:] = jnp.dot(xv, wq_ref[:], preferred_element_type=jnp.float32)

        qb = lax.broadcasted_iota(jnp.int32, (SQ, SKV), 0) // 64
        kb = lax.broadcasted_iota(jnp.int32, (SQ, SKV), 1) // 64
        mask = (qb == kb) | (kb == 0) | ((qb + kb) % 3 == 0)

        for b in range(B):
            for h in range(H_LOC):
                qh = q_ref[b * SQ:(b + 1) * SQ, h * DH:(h + 1) * DH]
                kh = k_ref[b, h]
                scores = lax.dot_general(
                    qh, kh, (((1,), (1,)), ((), ())),
                    preferred_element_type=jnp.float32,
                ) * 0.125
                scores = jnp.where(mask, scores, -1e9)
                m = jnp.max(scores, axis=1, keepdims=True)
                e = jnp.exp(scores - m)
                w = e / jnp.sum(e, axis=1, keepdims=True)
                ctx_ref[:, h * DH:(h + 1) * DH] = jnp.dot(
                    w, v_ref[b, h], preferred_element_type=jnp.float32
                )
            partial_b = jnp.dot(
                ctx_ref[:], wo_ref[:], preferred_element_type=jnp.float32
            )
            out_ref[b] = partial_b
            comm_ref[0, b] = partial_b

        for h in range(N_DEV - 1):
            send_slot = h % 2
            recv_slot = (h + 1) % 2
            rdma = pltpu.make_async_remote_copy(
                src_ref=comm_ref.at[send_slot],
                dst_ref=comm_ref.at[recv_slot],
                send_sem=send_sems.at[send_slot],
                recv_sem=recv_sems.at[recv_slot],
                device_id=(right,),
                device_id_type=pl.DeviceIdType.MESH,
            )
            rdma.start()
            rdma.wait()
            out_ref[:] = out_ref[:] + comm_ref[recv_slot]

    return pl.pallas_call(
        body,
        out_shape=jax.ShapeDtypeStruct((B, SQ, D_MODEL), jnp.float32),
        in_specs=[pl.BlockSpec(memory_space=pltpu.VMEM)] * 5,
        out_specs=pl.BlockSpec(memory_space=pltpu.VMEM),
        scratch_shapes=[
            pltpu.VMEM((B * SQ, HD_LOC), jnp.float32),
            pltpu.VMEM((SQ, HD_LOC), jnp.float32),
            pltpu.VMEM((2, B, SQ, D_MODEL), jnp.float32),
            pltpu.SemaphoreType.DMA((2,)),
            pltpu.SemaphoreType.DMA((2,)),
        ],
        compiler_params=pltpu.CompilerParams(collective_id=0),
    )(x, Wq_loc, Kt, Vt, Wo_loc)


# device time: 91782 ns/iter; 2.9798x vs baseline; 2.9798x over previous
import jax
import jax.numpy as jnp
from jax import lax
from jax.experimental import pallas as pl
from jax.experimental.pallas import tpu as pltpu

N_DEV = 8
B = 2
SQ = 512
SKV = 512
H_LOC = 8
DH = 64
D_MODEL = 768
HD_LOC = H_LOC * DH


def kernel(x, Wq, K_ext, V_ext, Wo):
    my = lax.axis_index("i")

    Wq_loc = lax.dynamic_slice_in_dim(Wq, my * HD_LOC, HD_LOC, axis=1)
    Wo_loc = lax.dynamic_slice_in_dim(Wo, my * HD_LOC, HD_LOC, axis=0)
    Kt = jnp.transpose(K_ext, (0, 2, 1, 3))
    Vt = jnp.transpose(V_ext, (0, 2, 1, 3))

    def body(x_ref, wq_ref, k_ref, v_ref, wo_ref, out_ref,
             q_ref, ctx_ref, w_ref, stg_ref, send_sems, recv_sems):
        pos = lax.axis_index("i")
        zbit = pos // 4
        q4 = pos % 4
        ybit = q4 // 2
        xbit = (q4 % 2) ^ ybit
        px = pos ^ 1
        py = pos ^ 3
        pz = pos ^ 4

        barrier_sem = pltpu.get_barrier_semaphore()
        for nbr in [px, py, pz]:
            pl.semaphore_signal(
                barrier_sem, inc=1,
                device_id=(nbr,), device_id_type=pl.DeviceIdType.MESH,
            )
        pl.semaphore_wait(barrier_sem, 3)

        xv = x_ref[:].reshape(B * SQ, D_MODEL)
        q_ref[:] = jnp.dot(xv, wq_ref[:], preferred_element_type=jnp.float32)

        qb = lax.broadcasted_iota(jnp.int32, (SQ, SKV), 0) // 64
        kb = lax.broadcasted_iota(jnp.int32, (SQ, SKV), 1) // 64
        mask = (qb == kb) | (kb == 0) | ((qb + kb) % 3 == 0)

        for b in range(B):
            for h in range(H_LOC):
                qh = q_ref[b * SQ:(b + 1) * SQ, h * DH:(h + 1) * DH]
                kh = k_ref[b, h]
                scores = lax.dot_general(
                    qh, kh, (((1,), (1,)), ((), ())),
                    preferred_element_type=jnp.float32,
                ) * 0.125
                scores = jnp.where(mask, scores, -1e9)
                m = jnp.max(scores, axis=1, keepdims=True)
                e = jnp.exp(scores - m)
                w = e / jnp.sum(e, axis=1, keepdims=True)
                ctx_ref[:, h * DH:(h + 1) * DH] = jnp.dot(
                    w, v_ref[b, h], preferred_element_type=jnp.float32
                )
            w_ref[b * SQ:(b + 1) * SQ, :] = jnp.dot(
                ctx_ref[:], wo_ref[:], preferred_element_type=jnp.float32
            )

        rounds = [
            (pz, zbit, 512, "rs"),
            (py, ybit, 256, "rs"),
            (px, xbit, 128, "rs"),
            (px, xbit, 128, "ag"),
            (py, ybit, 256, "ag"),
            (pz, zbit, 512, "ag"),
        ]
        lo = jnp.int32(0)
        for r, (pn, side, half, kind) in enumerate(rounds):
            if kind == "rs":
                send_off = lo + (1 - side) * half
                keep_off = lo + side * half
                rdma = pltpu.make_async_remote_copy(
                    src_ref=w_ref.at[pl.ds(send_off, half)],
                    dst_ref=stg_ref.at[r, pl.ds(0, half)],
                    send_sem=send_sems.at[r],
                    recv_sem=recv_sems.at[r],
                    device_id=(pn,),
                    device_id_type=pl.DeviceIdType.MESH,
                )
                rdma.start()
                rdma.wait_recv()
                w_ref[pl.ds(keep_off, half), :] = (
                    w_ref[pl.ds(keep_off, half), :] + stg_ref[r, 0:half, :]
                )
                rdma.wait_send()
                lo = keep_off
            else:
                rdma = pltpu.make_async_remote_copy(
                    src_ref=w_ref.at[pl.ds(lo, half)],
                    dst_ref=w_ref.at[pl.ds(lo, half)],
                    send_sem=send_sems.at[r],
                    recv_sem=recv_sems.at[r],
                    device_id=(pn,),
                    device_id_type=pl.DeviceIdType.MESH,
                )
                rdma.start()
                rdma.wait_recv()
                rdma.wait_send()
                lo = lo - side * half

        out_ref[0] = w_ref[0:SQ, :]
        out_ref[1] = w_ref[SQ:2 * SQ, :]

    return pl.pallas_call(
        body,
        out_shape=jax.ShapeDtypeStruct((B, SQ, D_MODEL), jnp.float32),
        in_specs=[pl.BlockSpec(memory_space=pltpu.VMEM)] * 5,
        out_specs=pl.BlockSpec(memory_space=pltpu.VMEM),
        scratch_shapes=[
            pltpu.VMEM((B * SQ, HD_LOC), jnp.float32),
            pltpu.VMEM((SQ, HD_LOC), jnp.float32),
            pltpu.VMEM((B * SQ, D_MODEL), jnp.float32),
            pltpu.VMEM((3, SQ, D_MODEL), jnp.float32),
            pltpu.SemaphoreType.DMA((6,)),
            pltpu.SemaphoreType.DMA((6,)),
        ],
        compiler_params=pltpu.CompilerParams(collective_id=0),
    )(x, Wq_loc, Kt, Vt, Wo_loc)


# device time: 86494 ns/iter; 3.1620x vs baseline; 1.0611x over previous
import jax
import jax.numpy as jnp
from jax import lax
from jax.experimental import pallas as pl
from jax.experimental.pallas import tpu as pltpu

N_DEV = 8
B = 2
SQ = 512
SKV = 512
H_LOC = 8
DH = 64
D_MODEL = 768
HD_LOC = H_LOC * DH


def kernel(x, Wq, K_ext, V_ext, Wo):
    my = lax.axis_index("i")

    Wq_loc = lax.dynamic_slice_in_dim(Wq, my * HD_LOC, HD_LOC, axis=1)
    Wo_loc = lax.dynamic_slice_in_dim(Wo, my * HD_LOC, HD_LOC, axis=0)
    Kt = jnp.transpose(K_ext, (0, 2, 1, 3)).reshape(B * H_LOC, SKV, DH)
    Vt = jnp.transpose(V_ext, (0, 2, 1, 3)).reshape(B * H_LOC, SKV, DH)

    def body(x_ref, wq_ref, k_ref, v_ref, wo_ref, out_ref,
             q_ref, ctx_ref, w_ref, stg_ref, send_sems, recv_sems):
        pos = lax.axis_index("i")
        zbit = pos // 4
        q4 = pos % 4
        ybit = q4 // 2
        xbit = (q4 % 2) ^ ybit
        px = pos ^ 1
        py = pos ^ 3
        pz = pos ^ 4

        barrier_sem = pltpu.get_barrier_semaphore()
        for nbr in [px, py, pz]:
            pl.semaphore_signal(
                barrier_sem, inc=1,
                device_id=(nbr,), device_id_type=pl.DeviceIdType.MESH,
            )
        pl.semaphore_wait(barrier_sem, 3)

        xv = x_ref[:].reshape(B * SQ, D_MODEL)
        q_ref[:] = jnp.dot(xv, wq_ref[:], preferred_element_type=jnp.float32)

        qb = lax.broadcasted_iota(jnp.int32, (SQ, SKV), 0) // 64
        kb = lax.broadcasted_iota(jnp.int32, (SQ, SKV), 1) // 64
        mask = (qb == kb) | (kb == 0) | ((qb + kb) % 3 == 0)

        def compute_batch(bidx):
            for h in range(H_LOC):
                qh = q_ref[pl.ds(bidx * SQ, SQ), h * DH:(h + 1) * DH]
                kh = k_ref[pl.ds(bidx * H_LOC + h, 1)][0]
                scores = lax.dot_general(
                    qh, kh, (((1,), (1,)), ((), ())),
                    preferred_element_type=jnp.float32,
                ) * 0.125
                scores = jnp.where(mask, scores, -1e9)
                m = jnp.max(scores, axis=1, keepdims=True)
                e = jnp.exp(scores - m)
                w = e / jnp.sum(e, axis=1, keepdims=True)
                vh = v_ref[pl.ds(bidx * H_LOC + h, 1)][0]
                ctx_ref[:, h * DH:(h + 1) * DH] = jnp.dot(
                    w, vh, preferred_element_type=jnp.float32
                )
            w_ref[pl.ds(bidx * SQ, SQ), :] = jnp.dot(
                ctx_ref[:], wo_ref[:], preferred_element_type=jnp.float32
            )

        b_send = 1 - zbit
        b_keep = zbit
        compute_batch(b_send)
        rdma_z = pltpu.make_async_remote_copy(
            src_ref=w_ref.at[pl.ds(b_send * SQ, SQ)],
            dst_ref=stg_ref.at[0],
            send_sem=send_sems.at[0],
            recv_sem=recv_sems.at[0],
            device_id=(pz,),
            device_id_type=pl.DeviceIdType.MESH,
        )
        rdma_z.start()
        compute_batch(b_keep)
        rdma_z.wait_recv()
        w_ref[pl.ds(b_keep * SQ, SQ), :] = (
            w_ref[pl.ds(b_keep * SQ, SQ), :] + stg_ref[0]
        )
        rdma_z.wait_send()

        rounds = [
            (py, ybit, 256, "rs"),
            (px, xbit, 128, "rs"),
            (px, xbit, 128, "ag"),
            (py, ybit, 256, "ag"),
            (pz, zbit, 512, "ag"),
        ]
        lo = zbit * 512
        for r0, (pn, side, half, kind) in enumerate(rounds):
            r = r0 + 1
            if kind == "rs":
                send_off = lo + (1 - side) * half
                keep_off = lo + side * half
                rdma = pltpu.make_async_remote_copy(
                    src_ref=w_ref.at[pl.ds(send_off, half)],
                    dst_ref=stg_ref.at[r, pl.ds(0, half)],
                    send_sem=send_sems.at[r],
                    recv_sem=recv_sems.at[r],
                    device_id=(pn,),
                    device_id_type=pl.DeviceIdType.MESH,
                )
                rdma.start()
                rdma.wait_recv()
                w_ref[pl.ds(keep_off, half), :] = (
                    w_ref[pl.ds(keep_off, half), :] + stg_ref[r, 0:half, :]
                )
                rdma.wait_send()
                lo = keep_off
            else:
                rdma = pltpu.make_async_remote_copy(
                    src_ref=w_ref.at[pl.ds(lo, half)],
                    dst_ref=w_ref.at[pl.ds(lo, half)],
                    send_sem=send_sems.at[r],
                    recv_sem=recv_sems.at[r],
                    device_id=(pn,),
                    device_id_type=pl.DeviceIdType.MESH,
                )
                rdma.start()
                rdma.wait_recv()
                rdma.wait_send()
                lo = lo - side * half

        out_ref[0] = w_ref[0:SQ, :]
        out_ref[1] = w_ref[SQ:2 * SQ, :]

    return pl.pallas_call(
        body,
        out_shape=jax.ShapeDtypeStruct((B, SQ, D_MODEL), jnp.float32),
        in_specs=[pl.BlockSpec(memory_space=pltpu.VMEM)] * 5,
        out_specs=pl.BlockSpec(memory_space=pltpu.VMEM),
        scratch_shapes=[
            pltpu.VMEM((B * SQ, HD_LOC), jnp.float32),
            pltpu.VMEM((SQ, HD_LOC), jnp.float32),
            pltpu.VMEM((B * SQ, D_MODEL), jnp.float32),
            pltpu.VMEM((3, SQ, D_MODEL), jnp.float32),
            pltpu.SemaphoreType.DMA((6,)),
            pltpu.SemaphoreType.DMA((6,)),
        ],
        compiler_params=pltpu.CompilerParams(collective_id=0),
    )(x, Wq_loc, Kt, Vt, Wo_loc)


# device time: 72750 ns/iter; 3.7593x vs baseline; 1.1889x over previous
import jax
import jax.numpy as jnp
from jax import lax
from jax.experimental import pallas as pl
from jax.experimental.pallas import tpu as pltpu

N_DEV = 8
B = 2
SQ = 512
SKV = 512
H_LOC = 8
DH = 64
D_MODEL = 768
HD_LOC = H_LOC * DH


def kernel(x, Wq, K_ext, V_ext, Wo):
    my = lax.axis_index("i")

    Wq_loc = lax.dynamic_slice_in_dim(Wq, my * HD_LOC, HD_LOC, axis=1)
    Wo_loc = lax.dynamic_slice_in_dim(Wo, my * HD_LOC, HD_LOC, axis=0)
    Kt = jnp.transpose(K_ext, (0, 2, 1, 3)).reshape(B * H_LOC, SKV, DH)
    Vt = jnp.transpose(V_ext, (0, 2, 1, 3)).reshape(B * H_LOC, SKV, DH)

    def body(x_ref, wq_ref, k_ref, v_ref, wo_ref, out_ref,
             q_ref, ctx_ref, w_ref, stg_ref, send_sems, recv_sems,
             ag_send_sems, ag_recv_sems):
        pos = lax.axis_index("i")
        zbit = pos // 4
        q4 = pos % 4
        ybit = q4 // 2
        xbit = (q4 % 2) ^ ybit
        px = pos ^ 1
        py = pos ^ 3
        pz = pos ^ 4

        barrier_sem = pltpu.get_barrier_semaphore()
        for nbr in [px, py, pz]:
            pl.semaphore_signal(
                barrier_sem, inc=1,
                device_id=(nbr,), device_id_type=pl.DeviceIdType.MESH,
            )
        pl.semaphore_wait(barrier_sem, 3)

        xv = x_ref[:].reshape(B * SQ, D_MODEL)
        q_ref[:] = jnp.dot(xv, wq_ref[:], preferred_element_type=jnp.float32)

        qb = lax.broadcasted_iota(jnp.int32, (SQ, SKV), 0) // 64
        kb = lax.broadcasted_iota(jnp.int32, (SQ, SKV), 1) // 64
        mask = (qb == kb) | (kb == 0) | ((qb + kb) % 3 == 0)

        def compute_batch(bidx):
            for h in range(H_LOC):
                qh = q_ref[pl.ds(bidx * SQ, SQ), h * DH:(h + 1) * DH]
                kh = k_ref[pl.ds(bidx * H_LOC + h, 1)][0]
                scores = lax.dot_general(
                    qh, kh, (((1,), (1,)), ((), ())),
                    preferred_element_type=jnp.float32,
                ) * 0.125
                scores = jnp.where(mask, scores, -1e9)
                m = jnp.max(scores, axis=1, keepdims=True)
                e = jnp.exp(scores - m)
                w = e / jnp.sum(e, axis=1, keepdims=True)
                vh = v_ref[pl.ds(bidx * H_LOC + h, 1)][0]
                ctx_ref[:, h * DH:(h + 1) * DH] = jnp.dot(
                    w, vh, preferred_element_type=jnp.float32
                )
            w_ref[pl.ds(bidx * SQ, SQ), :] = jnp.dot(
                ctx_ref[:], wo_ref[:], preferred_element_type=jnp.float32
            )

        b_send = 1 - zbit
        b_keep = zbit
        compute_batch(b_send)
        rdma_z = pltpu.make_async_remote_copy(
            src_ref=w_ref.at[pl.ds(b_send * SQ, SQ)],
            dst_ref=stg_ref.at[0],
            send_sem=send_sems.at[0],
            recv_sem=recv_sems.at[0],
            device_id=(pz,),
            device_id_type=pl.DeviceIdType.MESH,
        )
        rdma_z.start()
        compute_batch(b_keep)
        rdma_z.wait_recv()
        w_ref[pl.ds(b_keep * SQ, SQ), :] = (
            w_ref[pl.ds(b_keep * SQ, SQ), :] + stg_ref[0]
        )
        rdma_z.wait_send()

        lo = zbit * 512
        for r, (pn, side, half) in [(1, (py, ybit, 256)), (2, (px, xbit, 128))]:
            send_off = lo + (1 - side) * half
            keep_off = lo + side * half
            rdma = pltpu.make_async_remote_copy(
                src_ref=w_ref.at[pl.ds(send_off, half)],
                dst_ref=stg_ref.at[r, pl.ds(0, half)],
                send_sem=send_sems.at[r],
                recv_sem=recv_sems.at[r],
                device_id=(pn,),
                device_id_type=pl.DeviceIdType.MESH,
            )
            rdma.start()
            rdma.wait_recv()
            w_ref[pl.ds(keep_off, half), :] = (
                w_ref[pl.ds(keep_off, half), :] + stg_ref[r, 0:half, :]
            )
            rdma.wait_send()
            lo = keep_off

        L = lo // 128
        CH = 128

        def chunk_send(lbl, dest, idx):
            r = pltpu.make_async_remote_copy(
                src_ref=w_ref.at[pl.ds(lbl * CH, CH)],
                dst_ref=w_ref.at[pl.ds(lbl * CH, CH)],
                send_sem=ag_send_sems.at[idx],
                recv_sem=ag_recv_sems.at[idx],
                device_id=(dest,),
                device_id_type=pl.DeviceIdType.MESH,
            )
            r.start()
            return r

        def chunk_recv_wait(lbl, idx):
            pltpu.make_async_remote_copy(
                src_ref=w_ref.at[pl.ds(lbl * CH, CH)],
                dst_ref=w_ref.at[pl.ds(lbl * CH, CH)],
                send_sem=ag_send_sems.at[idx],
                recv_sem=ag_recv_sems.at[idx],
                device_id=(px,),
                device_id_type=pl.DeviceIdType.MESH,
            ).wait_recv()

        sends = [
            chunk_send(L, px, 0),
            chunk_send(L, py, 1),
            chunk_send(L, pz, 3),
        ]
        chunk_recv_wait(L ^ 1, 0)
        sends.append(chunk_send(L ^ 1, py, 2))
        sends.append(chunk_send(L ^ 1, pz, 4))
        chunk_recv_wait(L ^ 2, 1)
        sends.append(chunk_send(L ^ 2, pz, 5))
        chunk_recv_wait(L ^ 3, 2)
        sends.append(chunk_send(L ^ 3, pz, 6))
        for i in range(3, 7):
            chunk_recv_wait(L ^ (i + 1), i)
        for s in sends:
            s.wait_send()

        out_ref[0] = w_ref[0:SQ, :]
        out_ref[1] = w_ref[SQ:2 * SQ, :]

    return pl.pallas_call(
        body,
        out_shape=jax.ShapeDtypeStruct((B, SQ, D_MODEL), jnp.float32),
        in_specs=[pl.BlockSpec(memory_space=pltpu.VMEM)] * 5,
        out_specs=pl.BlockSpec(memory_space=pltpu.VMEM),
        scratch_shapes=[
            pltpu.VMEM((B * SQ, HD_LOC), jnp.float32),
            pltpu.VMEM((SQ, HD_LOC), jnp.float32),
            pltpu.VMEM((B * SQ, D_MODEL), jnp.float32),
            pltpu.VMEM((3, SQ, D_MODEL), jnp.float32),
            pltpu.SemaphoreType.DMA((3,)),
            pltpu.SemaphoreType.DMA((3,)),
            pltpu.SemaphoreType.DMA((7,)),
            pltpu.SemaphoreType.DMA((7,)),
        ],
        compiler_params=pltpu.CompilerParams(collective_id=0),
    )(x, Wq_loc, Kt, Vt, Wo_loc)


# device time: 62530 ns/iter; 4.3738x vs baseline; 1.1634x over previous
import jax
import jax.numpy as jnp
from jax import lax
from jax.experimental import pallas as pl
from jax.experimental.pallas import tpu as pltpu

N_DEV = 8
B = 2
SQ = 512
SKV = 512
H_LOC = 8
DH = 64
D_MODEL = 768
HD_LOC = H_LOC * DH


def kernel(x, Wq, K_ext, V_ext, Wo):
    my = lax.axis_index("i")

    Wq_loc = lax.dynamic_slice_in_dim(Wq, my * HD_LOC, HD_LOC, axis=1)
    Wo_loc = lax.dynamic_slice_in_dim(Wo, my * HD_LOC, HD_LOC, axis=0)
    Kt = jnp.transpose(K_ext, (0, 2, 1, 3)).reshape(B * H_LOC, SKV, DH)
    Vt = jnp.transpose(V_ext, (0, 2, 1, 3)).reshape(B * H_LOC, SKV, DH)

    def body(x_ref, wq_ref, k_ref, v_ref, wo_ref, out_ref,
             q_ref, ctx_ref, w_ref, stg_ref, send_sems, recv_sems):
        pos = lax.axis_index("i")
        zbit = pos // 4
        q4 = pos % 4
        ybit = q4 // 2
        xbit = (q4 % 2) ^ ybit
        px = pos ^ 1
        py = pos ^ 3
        pz = pos ^ 4

        barrier_sem = pltpu.get_barrier_semaphore()
        for nbr in [px, py, pz]:
            pl.semaphore_signal(
                barrier_sem, inc=1,
                device_id=(nbr,), device_id_type=pl.DeviceIdType.MESH,
            )
        pl.semaphore_wait(barrier_sem, 3)

        xv = x_ref[:].reshape(B * SQ, D_MODEL)
        q_ref[:] = jnp.dot(xv, wq_ref[:], preferred_element_type=jnp.float32)

        qb = lax.broadcasted_iota(jnp.int32, (SQ, SKV), 0) // 64
        kb = lax.broadcasted_iota(jnp.int32, (SQ, SKV), 1) // 64
        mask = (qb == kb) | (kb == 0) | ((qb + kb) % 3 == 0)

        def compute_batch(b):
            for h in range(H_LOC):
                qh = q_ref[b * SQ:(b + 1) * SQ, h * DH:(h + 1) * DH]
                kh = k_ref[b * H_LOC + h]
                scores = lax.dot_general(
                    qh, kh, (((1,), (1,)), ((), ())),
                    preferred_element_type=jnp.float32,
                ) * 0.125
                scores = jnp.where(mask, scores, -1e9)
                m = jnp.max(scores, axis=1, keepdims=True)
                e = jnp.exp(scores - m)
                w = e / jnp.sum(e, axis=1, keepdims=True)
                ctx_ref[:, h * DH:(h + 1) * DH] = jnp.dot(
                    w, v_ref[b * H_LOC + h], preferred_element_type=jnp.float32
                )
            w_ref[b * SQ:(b + 1) * SQ, :] = jnp.dot(
                ctx_ref[:], wo_ref[:], preferred_element_type=jnp.float32
            )

        exchanges = {
            "A1": (pz, zbit, 256, 0),
            "A2": (py, ybit, 128, 1),
            "A3": (px, xbit, 64, 2),
            "B1": (px, xbit, 256, 3),
            "B2": (pz, zbit, 128, 4),
            "B3": (py, ybit, 64, 5),
            "A4": (px, xbit, 64, 6),
            "A5": (py, ybit, 128, 7),
            "A6": (pz, zbit, 256, 8),
            "B4": (py, ybit, 64, 9),
            "B5": (pz, zbit, 128, 10),
            "B6": (px, xbit, 256, 11),
        }
        lo = {"A": jnp.int32(0), "B": jnp.int32(512)}
        live = {}

        def rs_start(name):
            part = name[0]
            pn, side, half, i = exchanges[name]
            send_off = lo[part] + (1 - side) * half
            rdma = pltpu.make_async_remote_copy(
                src_ref=w_ref.at[pl.ds(send_off, half)],
                dst_ref=stg_ref.at[i, pl.ds(0, half)],
                send_sem=send_sems.at[i],
                recv_sem=recv_sems.at[i],
                device_id=(pn,),
                device_id_type=pl.DeviceIdType.MESH,
            )
            rdma.start()
            live[name] = rdma

        def rs_finish(name):
            part = name[0]
            _, side, half, i = exchanges[name]
            live[name].wait_recv()
            keep_off = lo[part] + side * half
            w_ref[pl.ds(keep_off, half), :] = (
                w_ref[pl.ds(keep_off, half), :] + stg_ref[i, 0:half, :]
            )
            lo[part] = keep_off

        def ag_start(name):
            part = name[0]
            pn, side, half, i = exchanges[name]
            rdma = pltpu.make_async_remote_copy(
                src_ref=w_ref.at[pl.ds(lo[part], half)],
                dst_ref=w_ref.at[pl.ds(lo[part], half)],
                send_sem=send_sems.at[i],
                recv_sem=recv_sems.at[i],
                device_id=(pn,),
                device_id_type=pl.DeviceIdType.MESH,
            )
            rdma.start()
            live[name] = rdma

        def ag_finish(name):
            part = name[0]
            _, side, half, i = exchanges[name]
            live[name].wait_recv()
            lo[part] = lo[part] - side * half

        compute_batch(0)
        rs_start("A1")
        compute_batch(1)
        rs_start("B1")
        rs_finish("A1")
        rs_start("A2")
        rs_finish("B1")
        rs_start("B2")
        rs_finish("A2")
        rs_start("A3")
        rs_finish("B2")
        rs_start("B3")
        rs_finish("A3")
        ag_start("A4")
        rs_finish("B3")
        ag_start("B4")
        ag_finish("A4")
        ag_start("A5")
        ag_finish("B4")
        ag_start("B5")
        ag_finish("A5")
        ag_start("A6")
        ag_finish("B5")
        ag_start("B6")
        ag_finish("A6")
        ag_finish("B6")
        for rdma in live.values():
            rdma.wait_send()

        out_ref[0] = w_ref[0:SQ, :]
        out_ref[1] = w_ref[SQ:2 * SQ, :]

    return pl.pallas_call(
        body,
        out_shape=jax.ShapeDtypeStruct((B, SQ, D_MODEL), jnp.float32),
        in_specs=[pl.BlockSpec(memory_space=pltpu.VMEM)] * 5,
        out_specs=pl.BlockSpec(memory_space=pltpu.VMEM),
        scratch_shapes=[
            pltpu.VMEM((B * SQ, HD_LOC), jnp.float32),
            pltpu.VMEM((SQ, HD_LOC), jnp.float32),
            pltpu.VMEM((B * SQ, D_MODEL), jnp.float32),
            pltpu.VMEM((6, 256, D_MODEL), jnp.float32),
            pltpu.SemaphoreType.DMA((12,)),
            pltpu.SemaphoreType.DMA((12,)),
        ],
        compiler_params=pltpu.CompilerParams(collective_id=0),
    )(x, Wq_loc, Kt, Vt, Wo_loc)


# device time: 62427 ns/iter; 4.3810x vs baseline; 1.0016x over previous
import jax
import jax.numpy as jnp
from jax import lax
from jax.experimental import pallas as pl
from jax.experimental.pallas import tpu as pltpu

N_DEV = 8
B = 2
SQ = 512
SKV = 512
H_LOC = 8
DH = 64
D_MODEL = 768
HD_LOC = H_LOC * DH


def kernel(x, Wq, K_ext, V_ext, Wo):
    my = lax.axis_index("i")

    Wq_loc = lax.dynamic_slice_in_dim(Wq, my * HD_LOC, HD_LOC, axis=1)
    Wo_loc = lax.dynamic_slice_in_dim(Wo, my * HD_LOC, HD_LOC, axis=0)
    Kt = jnp.transpose(K_ext, (0, 2, 1, 3)).reshape(B * H_LOC, SKV, DH)
    Vt = jnp.transpose(V_ext, (0, 2, 1, 3)).reshape(B * H_LOC, SKV, DH)

    def body(x_ref, wq_ref, k_ref, v_ref, wo_ref, out_ref,
             q_ref, ctx_ref, w_ref, stg_ref, send_sems, recv_sems):
        pos = lax.axis_index("i")
        zbit = pos // 4
        q4 = pos % 4
        ybit = q4 // 2
        xbit = (q4 % 2) ^ ybit
        px = pos ^ 1
        py = pos ^ 3
        pz = pos ^ 4

        barrier_sem = pltpu.get_barrier_semaphore()
        for nbr in [px, py, pz]:
            pl.semaphore_signal(
                barrier_sem, inc=1,
                device_id=(nbr,), device_id_type=pl.DeviceIdType.MESH,
            )
        pl.semaphore_wait(barrier_sem, 3)

        xv = x_ref[:].reshape(B * SQ, D_MODEL)
        q_ref[:] = jnp.dot(xv, wq_ref[:], preferred_element_type=jnp.float32)

        qb = lax.broadcasted_iota(jnp.int32, (SQ, SKV), 0) // 64
        kb = lax.broadcasted_iota(jnp.int32, (SQ, SKV), 1) // 64
        mask = (qb == kb) | (kb == 0) | ((qb + kb) % 3 == 0)

        def compute_batch(b):
            for h in range(H_LOC):
                qh = q_ref[b * SQ:(b + 1) * SQ, h * DH:(h + 1) * DH]
                kh = k_ref[b * H_LOC + h]
                scores = lax.dot_general(
                    qh, kh, (((1,), (1,)), ((), ())),
                    preferred_element_type=jnp.float32,
                ) * 0.125
                scores = jnp.where(mask, scores, -1e9)
                m = jnp.max(scores, axis=1, keepdims=True)
                e = jnp.exp(scores - m)
                w = e / jnp.sum(e, axis=1, keepdims=True)
                ctx_ref[:, h * DH:(h + 1) * DH] = jnp.dot(
                    w, v_ref[b * H_LOC + h], preferred_element_type=jnp.float32
                )
            w_ref[b * SQ:(b + 1) * SQ, :] = jnp.dot(
                ctx_ref[:], wo_ref[:], preferred_element_type=jnp.float32
            )

        exchanges = {
            "A1": (pz, zbit, 256, 0),
            "A2": (py, ybit, 128, 1),
            "A3": (px, xbit, 64, 2),
            "B1": (px, xbit, 256, 3),
            "B2": (pz, zbit, 128, 4),
            "B3": (py, ybit, 64, 5),
            "A4": (px, xbit, 64, 6),
            "A5": (py, ybit, 128, 7),
            "A6": (pz, zbit, 256, 8),
            "B4": (py, ybit, 64, 9),
            "B5": (pz, zbit, 128, 10),
            "B6": (px, xbit, 256, 11),
        }
        lo = {"A": jnp.int32(0), "B": jnp.int32(512)}
        live = {}

        def rs_start(name):
            part = name[0]
            pn, side, half, i = exchanges[name]
            send_off = lo[part] + (1 - side) * half
            rdma = pltpu.make_async_remote_copy(
                src_ref=w_ref.at[pl.ds(send_off, half)],
                dst_ref=stg_ref.at[i, pl.ds(0, half)],
                send_sem=send_sems.at[i],
                recv_sem=recv_sems.at[i],
                device_id=(pn,),
                device_id_type=pl.DeviceIdType.MESH,
            )
            rdma.start()
            live[name] = rdma

        def rs_finish(name):
            part = name[0]
            _, side, half, i = exchanges[name]
            live[name].wait_recv()
            keep_off = lo[part] + side * half
            w_ref[pl.ds(keep_off, half), :] = (
                w_ref[pl.ds(keep_off, half), :] + stg_ref[i, 0:half, :]
            )
            lo[part] = keep_off

        def ag_start(name):
            part = name[0]
            pn, side, half, i = exchanges[name]
            rdma = pltpu.make_async_remote_copy(
                src_ref=w_ref.at[pl.ds(lo[part], half)],
                dst_ref=w_ref.at[pl.ds(lo[part], half)],
                send_sem=send_sems.at[i],
                recv_sem=recv_sems.at[i],
                device_id=(pn,),
                device_id_type=pl.DeviceIdType.MESH,
            )
            rdma.start()
            live[name] = rdma

        def ag_finish(name):
            part = name[0]
            _, side, half, i = exchanges[name]
            live[name].wait_recv()
            lo[part] = lo[part] - side * half

        compute_batch(0)
        rs_start("A1")
        compute_batch(1)
        rs_start("B1")
        rs_finish("A1")
        rs_start("A2")
        rs_finish("B1")
        rs_start("B2")
        rs_finish("A2")
        rs_start("A3")
        rs_finish("B2")
        rs_start("B3")
        rs_finish("A3")
        ag_start("A4")
        rs_finish("B3")
        ag_start("B4")
        ag_finish("A4")
        ag_start("A5")
        ag_finish("B4")
        ag_start("B5")
        ag_finish("A5")
        ag_start("A6")
        ag_finish("B5")
        ag_start("B6")
        ag_finish("A6")
        out_ref[0] = w_ref[0:SQ, :]
        ag_finish("B6")
        out_ref[1] = w_ref[SQ:2 * SQ, :]
        for rdma in live.values():
            rdma.wait_send()

    return pl.pallas_call(
        body,
        out_shape=jax.ShapeDtypeStruct((B, SQ, D_MODEL), jnp.float32),
        in_specs=[pl.BlockSpec(memory_space=pltpu.VMEM)] * 5,
        out_specs=pl.BlockSpec(memory_space=pltpu.VMEM),
        scratch_shapes=[
            pltpu.VMEM((B * SQ, HD_LOC), jnp.float32),
            pltpu.VMEM((SQ, HD_LOC), jnp.float32),
            pltpu.VMEM((B * SQ, D_MODEL), jnp.float32),
            pltpu.VMEM((6, 256, D_MODEL), jnp.float32),
            pltpu.SemaphoreType.DMA((12,)),
            pltpu.SemaphoreType.DMA((12,)),
        ],
        compiler_params=pltpu.CompilerParams(collective_id=0),
    )(x, Wq_loc, Kt, Vt, Wo_loc)


# device time: 57182 ns/iter; 4.7828x vs baseline; 1.0917x over previous
import jax
import jax.numpy as jnp
from jax import lax
from jax.experimental import pallas as pl
from jax.experimental.pallas import tpu as pltpu

N_DEV = 8
B = 2
SQ = 512
SKV = 512
H_LOC = 8
DH = 64
D_MODEL = 768
HD_LOC = H_LOC * DH


def kernel(x, Wq, K_ext, V_ext, Wo):
    my = lax.axis_index("i")

    Wq_loc = lax.dynamic_slice_in_dim(Wq, my * HD_LOC, HD_LOC, axis=1)
    Wo_loc = lax.dynamic_slice_in_dim(Wo, my * HD_LOC, HD_LOC, axis=0)
    Kt = jnp.transpose(K_ext, (0, 2, 1, 3)).reshape(B * H_LOC, SKV, DH)
    Vt = jnp.transpose(V_ext, (0, 2, 1, 3)).reshape(B * H_LOC, SKV, DH)

    def body(x_ref, wq_ref, k_ref, v_ref, wo_ref, out_ref,
             q_ref, ctx_ref, w_ref, stg_ref, send_sems, recv_sems):
        pos = lax.axis_index("i")
        zbit = pos // 4
        q4 = pos % 4
        ybit = q4 // 2
        xbit = (q4 % 2) ^ ybit
        px = pos ^ 1
        py = pos ^ 3
        pz = pos ^ 4

        barrier_sem = pltpu.get_barrier_semaphore()
        for nbr in [px, py, pz]:
            pl.semaphore_signal(
                barrier_sem, inc=1,
                device_id=(nbr,), device_id_type=pl.DeviceIdType.MESH,
            )
        pl.semaphore_wait(barrier_sem, 3)

        xv = x_ref[:].reshape(B * SQ, D_MODEL)
        q_ref[:] = jnp.dot(xv, wq_ref[:], preferred_element_type=jnp.float32)

        qb = lax.broadcasted_iota(jnp.int32, (SQ, SKV), 0) // 64
        kb = lax.broadcasted_iota(jnp.int32, (SQ, SKV), 1) // 64
        mask = (qb == kb) | (kb == 0) | ((qb + kb) % 3 == 0)

        def compute_batch(b):
            for h in range(H_LOC):
                qh = q_ref[b * SQ:(b + 1) * SQ, h * DH:(h + 1) * DH]
                kh = k_ref[b * H_LOC + h]
                scores = lax.dot_general(
                    qh, kh, (((1,), (1,)), ((), ())),
                    preferred_element_type=jnp.float32,
                ) * 0.125
                scores = jnp.where(mask, scores, -1e9)
                m = jnp.max(scores, axis=1, keepdims=True)
                e = jnp.exp(scores - m)
                w = e / jnp.sum(e, axis=1, keepdims=True)
                ctx_ref[:, h * DH:(h + 1) * DH] = jnp.dot(
                    w, v_ref[b * H_LOC + h], preferred_element_type=jnp.float32
                )
            w_ref[b * SQ:(b + 1) * SQ, :] = jnp.dot(
                ctx_ref[:], wo_ref[:], preferred_element_type=jnp.float32
            )

        exchanges = {
            "A1": (py, ybit, 256, 0),
            "A2": (pz, zbit, 128, 1),
            "A3": (px, xbit, 64, 2),
            "B1": (px, xbit, 128, 3),
            "B2": (pz, zbit, 64, 4),
            "B3": (py, ybit, 32, 5),
            "C1": (pz, zbit, 128, 6),
            "C2": (px, xbit, 64, 7),
            "C3": (py, ybit, 32, 8),
            "A4": (px, xbit, 64, 9),
            "A5": (pz, zbit, 128, 10),
            "A6": (py, ybit, 256, 11),
            "B4": (py, ybit, 32, 12),
            "B5": (pz, zbit, 64, 13),
            "B6": (px, xbit, 128, 14),
            "C4": (py, ybit, 32, 15),
            "C5": (px, xbit, 64, 16),
            "C6": (pz, zbit, 128, 17),
        }
        lo = {"A": jnp.int32(0), "B": jnp.int32(512), "C": jnp.int32(768)}
        live = {}

        def rs_start(name):
            part = name[0]
            pn, side, half, i = exchanges[name]
            send_off = lo[part] + (1 - side) * half
            rdma = pltpu.make_async_remote_copy(
                src_ref=w_ref.at[pl.ds(send_off, half)],
                dst_ref=stg_ref.at[i, pl.ds(0, half)],
                send_sem=send_sems.at[i],
                recv_sem=recv_sems.at[i],
                device_id=(pn,),
                device_id_type=pl.DeviceIdType.MESH,
            )
            rdma.start()
            live[name] = rdma

        def rs_finish(name):
            part = name[0]
            _, side, half, i = exchanges[name]
            live[name].wait_recv()
            keep_off = lo[part] + side * half
            w_ref[pl.ds(keep_off, half), :] = (
                w_ref[pl.ds(keep_off, half), :] + stg_ref[i, 0:half, :]
            )
            lo[part] = keep_off

        def ag_start(name):
            part = name[0]
            pn, side, half, i = exchanges[name]
            rdma = pltpu.make_async_remote_copy(
                src_ref=w_ref.at[pl.ds(lo[part], half)],
                dst_ref=w_ref.at[pl.ds(lo[part], half)],
                send_sem=send_sems.at[i],
                recv_sem=recv_sems.at[i],
                device_id=(pn,),
                device_id_type=pl.DeviceIdType.MESH,
            )
            rdma.start()
            live[name] = rdma

        def ag_finish(name):
            part = name[0]
            _, side, half, i = exchanges[name]
            live[name].wait_recv()
            lo[part] = lo[part] - side * half

        compute_batch(0)
        rs_start("A1")
        compute_batch(1)
        rs_start("B1")
        rs_start("C1")
        rs_finish("A1")
        rs_start("A2")
        rs_finish("B1")
        rs_start("B2")
        rs_finish("C1")
        rs_start("C2")
        rs_finish("A2")
        rs_start("A3")
        rs_finish("B2")
        rs_start("B3")
        rs_finish("C2")
        rs_start("C3")
        rs_finish("A3")
        ag_start("A4")
        rs_finish("B3")
        ag_start("B4")
        rs_finish("C3")
        ag_start("C4")
        ag_finish("A4")
        ag_start("A5")
        ag_finish("B4")
        ag_start("B5")
        ag_finish("C4")
        ag_start("C5")
        ag_finish("A5")
        ag_start("A6")
        ag_finish("B5")
        ag_start("B6")
        ag_finish("C5")
        ag_start("C6")
        ag_finish("A6")
        out_ref[0] = w_ref[0:SQ, :]
        ag_finish("B6")
        ag_finish("C6")
        out_ref[1] = w_ref[SQ:2 * SQ, :]
        for rdma in live.values():
            rdma.wait_send()

    return pl.pallas_call(
        body,
        out_shape=jax.ShapeDtypeStruct((B, SQ, D_MODEL), jnp.float32),
        in_specs=[pl.BlockSpec(memory_space=pltpu.VMEM)] * 5,
        out_specs=pl.BlockSpec(memory_space=pltpu.VMEM),
        scratch_shapes=[
            pltpu.VMEM((B * SQ, HD_LOC), jnp.float32),
            pltpu.VMEM((SQ, HD_LOC), jnp.float32),
            pltpu.VMEM((B * SQ, D_MODEL), jnp.float32),
            pltpu.VMEM((9, 256, D_MODEL), jnp.float32),
            pltpu.SemaphoreType.DMA((18,)),
            pltpu.SemaphoreType.DMA((18,)),
        ],
        compiler_params=pltpu.CompilerParams(collective_id=0),
    )(x, Wq_loc, Kt, Vt, Wo_loc)
